# Initial kernel scaffold; baseline (speedup 1.0000x reference)
import jax
import jax.numpy as jnp
from jax.experimental import pallas as pl


def kernel(x, edge_index, W1, b1, W_mu, b_mu, W_logstd, b_logstd):
    raise NotImplementedError("write your pallas kernel here")


# speedup vs baseline: 15.2520x; 15.2520x over previous
import functools

import jax
import jax.numpy as jnp
from jax import lax
from jax.experimental import pallas as pl
from jax.experimental.pallas import tpu as pltpu
from jax.experimental.pallas import tpu_sc as plsc

N = 10000
NP = 10240
E = 320000
D_IN = 128
D_HID = 128
D_OUT = 64

NC = 2
NS = 16
NW = NC * NS
EPW = E // NW
K = 80
NCHUNK = EPW // K
RPT = NP // NS
ZR = 128

_mesh = plsc.VectorSubcoreMesh(core_axis_name="c", subcore_axis_name="s")


DW = 128


@functools.partial(
    pl.kernel,
    out_type=jax.ShapeDtypeStruct((NC, NP, DW), jnp.float32),
    scratch_types=[
        pltpu.VMEM((K,), jnp.int32),
        pltpu.VMEM((K, DW), jnp.float32),
        pltpu.VMEM((ZR, DW), jnp.float32),
        pltpu.VMEM_SHARED((NP, DW), jnp.float32),
    ],
    mesh=_mesh,
)
def _sc_deg(dst_hbm, out_hbm, dst_v, ones_v, zero_v, acc_sh):
    cid = lax.axis_index("c")
    sid = lax.axis_index("s")
    wid = sid * NC + cid

    def ofill(i, _):
        for j in range(DW // 16):
            ones_v[i, pl.ds(j * 16, 16)] = jnp.ones((16,), jnp.float32)
        return 0
    lax.fori_loop(0, K, ofill, 0)

    def zfill(i, _):
        for j in range(DW // 16):
            zero_v[i, pl.ds(j * 16, 16)] = jnp.zeros((16,), jnp.float32)
        return 0
    lax.fori_loop(0, ZR, zfill, 0)

    for t in range(RPT // ZR):
        pltpu.sync_copy(zero_v, acc_sh.at[pl.ds(sid * RPT + t * ZR, ZR)])
    plsc.subcore_barrier()

    ebase = wid * EPW

    def step(i, _):
        pltpu.sync_copy(dst_hbm.at[pl.ds(ebase + i * K, K)], dst_v)
        pltpu.sync_copy(ones_v, acc_sh.at[dst_v], add=True)
        return 0
    lax.fori_loop(0, NCHUNK, step, 0)

    plsc.subcore_barrier()
    for t in range(RPT // ZR):
        r = sid * RPT + t * ZR
        pltpu.sync_copy(acc_sh.at[pl.ds(r, ZR)],
                        out_hbm.at[cid, pl.ds(r, ZR)])


def _make_sc_scatter(d):
    @functools.partial(
        pl.kernel,
        out_type=jax.ShapeDtypeStruct((NC, NP, d), jnp.float32),
        scratch_types=[
            pltpu.VMEM((K,), jnp.int32),
            pltpu.VMEM((K,), jnp.int32),
            pltpu.VMEM((K, d), jnp.float32),
            pltpu.VMEM((ZR, d), jnp.float32),
            pltpu.VMEM_SHARED((NP, d), jnp.float32),
            pltpu.SemaphoreType.DMA,
        ],
        mesh=_mesh,
    )
    def _sc_scatter(v_hbm, src_hbm, dst_hbm, out_hbm,
                    src_v, dst_v, rows_v, zrow_v, acc_sh, sem):
        cid = lax.axis_index("c")
        sid = lax.axis_index("s")
        wid = sid * NC + cid

        def zfill(i, _):
            for j in range(d // 16):
                zrow_v[i, pl.ds(j * 16, 16)] = jnp.zeros((16,), jnp.float32)
            return 0
        lax.fori_loop(0, ZR, zfill, 0)

        for t in range(RPT // ZR):
            pltpu.sync_copy(zrow_v, acc_sh.at[pl.ds(sid * RPT + t * ZR, ZR)])
        plsc.subcore_barrier()

        ebase = wid * EPW

        def step(i, _):
            b = ebase + i * K
            pltpu.sync_copy(src_hbm.at[pl.ds(b, K)], src_v)
            pltpu.sync_copy(dst_hbm.at[pl.ds(b, K)], dst_v)
            pltpu.async_copy(v_hbm.at[src_v], rows_v, sem).wait()
            pltpu.sync_copy(rows_v, acc_sh.at[dst_v], add=True)
            return 0
        lax.fori_loop(0, NCHUNK, step, 0)

        plsc.subcore_barrier()
        for t in range(RPT // ZR):
            r = sid * RPT + t * ZR
            pltpu.sync_copy(acc_sh.at[pl.ds(r, ZR)],
                            out_hbm.at[cid, pl.ds(r, ZR)])

    return _sc_scatter


_sc_scatter_128 = _make_sc_scatter(D_IN)


_BN = 1000
_GRID = N // _BN


def _dinv_from(deg_ref):
    deg = deg_ref[0, :, 0] + deg_ref[1, :, 0] + 1.0
    return lax.rsqrt(deg)


def _tc1_body(deg_ref, x_ref, xs_ref):
    dinv = _dinv_from(deg_ref)
    xs_ref[...] = x_ref[...] * dinv[:, None]


def _tc2_body(deg_ref, s1_ref, xs_ref, w1_ref, b1_ref, hs_ref):
    dinv = _dinv_from(deg_ref)
    agg = (s1_ref[0] + s1_ref[1] + xs_ref[...]) * dinv[:, None]
    h = jnp.dot(agg, w1_ref[...], preferred_element_type=jnp.float32)
    h = jnp.maximum(h + b1_ref[...][None, :], 0.0)
    hs_ref[...] = h * dinv[:, None]


def _tc3_body(deg_ref, s2_ref, hs_ref, wm_ref, bm_ref, wl_ref, bl_ref,
              mu_ref, ls_ref):
    dinv = _dinv_from(deg_ref)
    g = (s2_ref[0] + s2_ref[1] + hs_ref[...]) * dinv[:, None]
    mu_ref[...] = (jnp.dot(g, wm_ref[...], preferred_element_type=jnp.float32)
                   + bm_ref[...][None, :])
    ls_ref[...] = (jnp.dot(g, wl_ref[...], preferred_element_type=jnp.float32)
                   + bl_ref[...][None, :])


_deg_spec = pl.BlockSpec((NC, _BN, 1), lambda i: (0, i, 0))
_row_spec = pl.BlockSpec((_BN, D_IN), lambda i: (i, 0))
_acc_spec = pl.BlockSpec((NC, _BN, D_IN), lambda i: (0, i, 0))


def _tc1(deg3, x):
    return pl.pallas_call(
        _tc1_body,
        grid=(_GRID,),
        in_specs=[_deg_spec, _row_spec],
        out_specs=_row_spec,
        out_shape=jax.ShapeDtypeStruct((N, D_IN), jnp.float32),
    )(deg3, x)


def _tc2(deg3, s1, xs, w1, b1):
    return pl.pallas_call(
        _tc2_body,
        grid=(_GRID,),
        in_specs=[
            _deg_spec, _acc_spec, _row_spec,
            pl.BlockSpec((D_IN, D_HID), lambda i: (0, 0)),
            pl.BlockSpec((D_HID,), lambda i: (0,)),
        ],
        out_specs=pl.BlockSpec((_BN, D_HID), lambda i: (i, 0)),
        out_shape=jax.ShapeDtypeStruct((N, D_HID), jnp.float32),
    )(deg3, s1, xs, w1, b1)


def _tc3(deg3, s2, hs, w_mu, b_mu, w_logstd, b_logstd):
    out_spec = pl.BlockSpec((_BN, D_OUT), lambda i: (i, 0))
    return pl.pallas_call(
        _tc3_body,
        grid=(_GRID,),
        in_specs=[
            _deg_spec, _acc_spec, pl.BlockSpec((_BN, D_HID), lambda i: (i, 0)),
            pl.BlockSpec((D_HID, D_OUT), lambda i: (0, 0)),
            pl.BlockSpec((D_OUT,), lambda i: (0,)),
            pl.BlockSpec((D_HID, D_OUT), lambda i: (0, 0)),
            pl.BlockSpec((D_OUT,), lambda i: (0,)),
        ],
        out_specs=[out_spec, out_spec],
        out_shape=[jax.ShapeDtypeStruct((N, D_OUT), jnp.float32),
                   jax.ShapeDtypeStruct((N, D_OUT), jnp.float32)],
    )(deg3, s2, hs, w_mu, b_mu, w_logstd, b_logstd)


def kernel(x, edge_index, W1, b1, W_mu, b_mu, W_logstd, b_logstd):
    src = edge_index[0]
    dst = edge_index[1]
    deg2 = _sc_deg(dst)
    deg3 = deg2[:, :, :1]
    xs = _tc1(deg3, x)
    s1 = _sc_scatter_128(xs, src, dst)
    hs = _tc2(deg3, s1, xs, W1, b1)
    s2 = _sc_scatter_128(hs, src, dst)
    mu, logstd = _tc3(deg3, s2, hs, W_mu, b_mu, W_logstd, b_logstd)
    return (mu, logstd)
